# Initial kernel scaffold; baseline (speedup 1.0000x reference)
import functools

import jax
import jax.numpy as jnp
from jax import lax
from jax.experimental import pallas as pl
from jax.experimental.pallas import tpu as pltpu
from jax.experimental.pallas import tpu_sc as plsc

D = 128
CHUNK = 128
NC = 2
NS = 16
NW = NC * NS
DEGW = 16


def _mesh():
    return plsc.VectorSubcoreMesh(core_axis_name="c", subcore_axis_name="s")


def _make_deg_kernel(n_pad, cpt):
    slab = n_pad // NS

    @functools.partial(
        pl.kernel,
        out_type=jax.ShapeDtypeStruct((NC, n_pad, DEGW), jnp.float32),
        mesh=_mesh(),
        scratch_types=[
            pltpu.VMEM((cpt, CHUNK), jnp.int32),
            pltpu.VMEM((CHUNK, DEGW), jnp.float32),
            pltpu.VMEM_SHARED((n_pad, DEGW), jnp.float32),
        ],
    )
    def deg_kernel(col_hbm, ones_hbm, zeros_hbm, out_hbm, idx_v, ones_v, deg_sh):
        cid = lax.axis_index("c")
        sid = lax.axis_index("s")
        gid = cid * NS + sid
        pltpu.sync_copy(zeros_hbm, deg_sh.at[pl.ds(sid * slab, slab)])
        pltpu.sync_copy(ones_hbm, ones_v)
        pltpu.sync_copy(col_hbm.at[pl.ds(gid * cpt, cpt)], idx_v)
        plsc.subcore_barrier()

        def body(j, carry):
            pltpu.sync_copy(ones_v, deg_sh.at[idx_v.at[j]], add=True)
            return carry

        lax.fori_loop(0, cpt, body, 0)
        plsc.subcore_barrier()
        pltpu.sync_copy(
            deg_sh.at[pl.ds(sid * slab, slab)],
            out_hbm.at[cid, pl.ds(sid * slab, slab)],
        )

    return deg_kernel


def _make_scatter_kernel(n_pad, cpt):
    slab = n_pad // NS

    @functools.partial(
        pl.kernel,
        out_type=jax.ShapeDtypeStruct((NC, n_pad, D), jnp.float32),
        mesh=_mesh(),
        scratch_types=[
            pltpu.VMEM((cpt, CHUNK), jnp.int32),
            pltpu.VMEM((cpt, CHUNK), jnp.int32),
            pltpu.VMEM((CHUNK, D), jnp.float32),
            pltpu.VMEM_SHARED((n_pad, D), jnp.float32),
        ],
    )
    def scatter_kernel(y_hbm, row_hbm, col_hbm, zeros_hbm, out_hbm,
                       rowidx_v, colidx_v, buf_v, acc_sh):
        cid = lax.axis_index("c")
        sid = lax.axis_index("s")
        gid = cid * NS + sid
        pltpu.sync_copy(zeros_hbm, acc_sh.at[pl.ds(sid * slab, slab)])
        pltpu.sync_copy(row_hbm.at[pl.ds(gid * cpt, cpt)], rowidx_v)
        pltpu.sync_copy(col_hbm.at[pl.ds(gid * cpt, cpt)], colidx_v)
        plsc.subcore_barrier()

        def body(j, carry):
            pltpu.sync_copy(y_hbm.at[rowidx_v.at[j]], buf_v)
            pltpu.sync_copy(buf_v, acc_sh.at[colidx_v.at[j]], add=True)
            return carry

        lax.fori_loop(0, cpt, body, 0)
        plsc.subcore_barrier()
        pltpu.sync_copy(
            acc_sh.at[pl.ds(sid * slab, slab)],
            out_hbm.at[cid, pl.ds(sid * slab, slab)],
        )

    return scatter_kernel


def _linear_body(x_ref, w_ref, degs_ref, y_ref, dis_ref):
    d16 = degs_ref[0] + degs_ref[1]
    deg = jnp.sum(d16, axis=1, keepdims=True) * (1.0 / DEGW) + 1.0
    dis = lax.rsqrt(deg)
    xw = jnp.dot(x_ref[...], w_ref[...], preferred_element_type=jnp.float32)
    y_ref[...] = xw * dis
    dis_ref[...] = dis


def _post_body(n, accs_ref, y_ref, dis_ref, b_ref, gamma_ref, beta_ref, out_ref):
    a = accs_ref[0] + accs_ref[1] + y_ref[...]
    pre = dis_ref[...][:n] * a[:n] + b_ref[...]
    act = jnp.where(pre >= 0, pre, 0.01 * pre)
    mean = jnp.mean(act, axis=0, keepdims=True)
    var = jnp.mean((act - mean) ** 2, axis=0, keepdims=True)
    out_ref[...] = (act - mean) * lax.rsqrt(var + 1e-5) * gamma_ref[...] + beta_ref[...]


def kernel(x, edge_index, W, b, gamma, beta):
    n, d_in = x.shape
    d_out = W.shape[1]
    e = edge_index.shape[1]
    n_pad = ((n + 1) + NS - 1) // NS * NS
    cpt = (e + NW * CHUNK - 1) // (NW * CHUNK)
    e_pad = NW * CHUNK * cpt

    ei = edge_index.astype(jnp.int32)
    pad = jnp.full((e_pad - e,), n, dtype=jnp.int32)
    row2d = jnp.concatenate([ei[0], pad]).reshape(-1, CHUNK)
    col2d = jnp.concatenate([ei[1], pad]).reshape(-1, CHUNK)
    x_pad = jnp.pad(x, ((0, n_pad - n), (0, 0)))

    slab = n_pad // NS
    ones16 = jnp.ones((CHUNK, DEGW), jnp.float32)
    zeros16 = jnp.zeros((slab, DEGW), jnp.float32)
    zerosd = jnp.zeros((slab, D), jnp.float32)

    degs = _make_deg_kernel(n_pad, cpt)(col2d, ones16, zeros16)

    y, dis = pl.pallas_call(
        _linear_body,
        out_shape=[
            jax.ShapeDtypeStruct((n_pad, d_out), jnp.float32),
            jax.ShapeDtypeStruct((n_pad, 1), jnp.float32),
        ],
    )(x_pad, W, degs)

    accs = _make_scatter_kernel(n_pad, cpt)(y, row2d, col2d, zerosd)

    out = pl.pallas_call(
        functools.partial(_post_body, n),
        out_shape=jax.ShapeDtypeStruct((n, d_out), jnp.float32),
    )(accs, y, dis, b.reshape(1, -1), gamma.reshape(1, -1), beta.reshape(1, -1))
    return out


# speedup vs baseline: 11.9290x; 11.9290x over previous
import functools

import jax
import jax.numpy as jnp
from jax import lax
from jax.experimental import pallas as pl
from jax.experimental.pallas import tpu as pltpu
from jax.experimental.pallas import tpu_sc as plsc

D = 128
CHUNK = 128
NC = 2
NS = 16
NW = NC * NS
DEGW = 128


def _mesh():
    return plsc.VectorSubcoreMesh(core_axis_name="c", subcore_axis_name="s")


def _make_deg_kernel(n_pad, cpt):
    slab = n_pad // NS

    @functools.partial(
        pl.kernel,
        out_type=jax.ShapeDtypeStruct((NC, n_pad, DEGW), jnp.float32),
        mesh=_mesh(),
        scratch_types=[
            pltpu.VMEM((cpt, CHUNK), jnp.int32),
            pltpu.VMEM((CHUNK, DEGW), jnp.float32),
            pltpu.VMEM_SHARED((n_pad, DEGW), jnp.float32),
        ],
    )
    def deg_kernel(col_hbm, ones_hbm, zeros_hbm, out_hbm, idx_v, ones_v, deg_sh):
        cid = lax.axis_index("c")
        sid = lax.axis_index("s")
        gid = cid * NS + sid
        pltpu.sync_copy(zeros_hbm, deg_sh.at[pl.ds(sid * slab, slab)])
        pltpu.sync_copy(ones_hbm, ones_v)
        pltpu.sync_copy(col_hbm.at[pl.ds(gid * cpt, cpt)], idx_v)
        plsc.subcore_barrier()

        def body(j, carry):
            pltpu.sync_copy(ones_v, deg_sh.at[idx_v.at[j]], add=True)
            return carry

        lax.fori_loop(0, cpt, body, 0)
        plsc.subcore_barrier()
        pltpu.sync_copy(
            deg_sh.at[pl.ds(sid * slab, slab)],
            out_hbm.at[cid, pl.ds(sid * slab, slab)],
        )

    return deg_kernel


def _make_scatter_kernel(n_pad, cpt):
    slab = n_pad // NS

    @functools.partial(
        pl.kernel,
        out_type=jax.ShapeDtypeStruct((NC, n_pad, D), jnp.float32),
        mesh=_mesh(),
        scratch_types=[
            pltpu.VMEM((cpt, CHUNK), jnp.int32),
            pltpu.VMEM((cpt, CHUNK), jnp.int32),
            pltpu.VMEM((CHUNK, D), jnp.float32),
            pltpu.VMEM_SHARED((n_pad, D), jnp.float32),
        ],
    )
    def scatter_kernel(y_hbm, row_hbm, col_hbm, zeros_hbm, out_hbm,
                       rowidx_v, colidx_v, buf_v, acc_sh):
        cid = lax.axis_index("c")
        sid = lax.axis_index("s")
        gid = cid * NS + sid
        pltpu.sync_copy(zeros_hbm, acc_sh.at[pl.ds(sid * slab, slab)])
        pltpu.sync_copy(row_hbm.at[pl.ds(gid * cpt, cpt)], rowidx_v)
        pltpu.sync_copy(col_hbm.at[pl.ds(gid * cpt, cpt)], colidx_v)
        plsc.subcore_barrier()

        def body(j, carry):
            pltpu.sync_copy(y_hbm.at[rowidx_v.at[j]], buf_v)
            pltpu.sync_copy(buf_v, acc_sh.at[colidx_v.at[j]], add=True)
            return carry

        lax.fori_loop(0, cpt, body, 0)
        plsc.subcore_barrier()
        pltpu.sync_copy(
            acc_sh.at[pl.ds(sid * slab, slab)],
            out_hbm.at[cid, pl.ds(sid * slab, slab)],
        )

    return scatter_kernel


def _linear_body(x_ref, w_ref, degs_ref, y_ref, dis_ref):
    d16 = degs_ref[0] + degs_ref[1]
    deg = jnp.sum(d16, axis=1, keepdims=True) * (1.0 / DEGW) + 1.0
    dis = lax.rsqrt(deg)
    xw = jnp.dot(x_ref[...], w_ref[...], preferred_element_type=jnp.float32)
    y_ref[...] = xw * dis
    dis_ref[...] = dis


def _post_body(n, accs_ref, y_ref, dis_ref, b_ref, gamma_ref, beta_ref, out_ref):
    a = accs_ref[0] + accs_ref[1] + y_ref[...]
    pre = dis_ref[...][:n] * a[:n] + b_ref[...]
    act = jnp.where(pre >= 0, pre, 0.01 * pre)
    mean = jnp.mean(act, axis=0, keepdims=True)
    var = jnp.mean((act - mean) ** 2, axis=0, keepdims=True)
    out_ref[...] = (act - mean) * lax.rsqrt(var + 1e-5) * gamma_ref[...] + beta_ref[...]


def kernel(x, edge_index, W, b, gamma, beta):
    n, d_in = x.shape
    d_out = W.shape[1]
    e = edge_index.shape[1]
    n_pad = ((n + 1) + 8 * NS - 1) // (8 * NS) * (8 * NS)
    cpt = (e + NW * CHUNK - 1) // (NW * CHUNK)
    cpt = (cpt + 7) // 8 * 8
    e_pad = NW * CHUNK * cpt

    ei = edge_index.astype(jnp.int32)
    pad = jnp.full((e_pad - e,), n, dtype=jnp.int32)
    row2d = jnp.concatenate([ei[0], pad]).reshape(-1, CHUNK)
    col2d = jnp.concatenate([ei[1], pad]).reshape(-1, CHUNK)
    x_pad = jnp.pad(x, ((0, n_pad - n), (0, 0)))

    slab = n_pad // NS
    ones16 = jnp.ones((CHUNK, DEGW), jnp.float32)
    zeros16 = jnp.zeros((slab, DEGW), jnp.float32)
    zerosd = jnp.zeros((slab, D), jnp.float32)

    degs = _make_deg_kernel(n_pad, cpt)(col2d, ones16, zeros16)

    y, dis = pl.pallas_call(
        _linear_body,
        out_shape=[
            jax.ShapeDtypeStruct((n_pad, d_out), jnp.float32),
            jax.ShapeDtypeStruct((n_pad, 1), jnp.float32),
        ],
    )(x_pad, W, degs)

    accs = _make_scatter_kernel(n_pad, cpt)(y, row2d, col2d, zerosd)

    out = pl.pallas_call(
        functools.partial(_post_body, n),
        out_shape=jax.ShapeDtypeStruct((n, d_out), jnp.float32),
    )(accs, y, dis, b.reshape(1, -1), gamma.reshape(1, -1), beta.reshape(1, -1))
    return out


# speedup vs baseline: 13.1687x; 1.1039x over previous
import functools

import jax
import jax.numpy as jnp
from jax import lax
from jax.experimental import pallas as pl
from jax.experimental.pallas import tpu as pltpu
from jax.experimental.pallas import tpu_sc as plsc

D = 128
CHUNK = 128
NC = 2
NS = 16
NW = NC * NS
IDXB = 16
DEGW = 128


def _mesh():
    return plsc.VectorSubcoreMesh(core_axis_name="c", subcore_axis_name="s")


def _make_deg_kernel(n_pad, cpt):
    slab = n_pad // NS

    @functools.partial(
        pl.kernel,
        out_type=jax.ShapeDtypeStruct((NC, n_pad, DEGW), jnp.float32),
        mesh=_mesh(),
        scratch_types=[
            pltpu.VMEM((cpt, CHUNK), jnp.int32),
            pltpu.VMEM((CHUNK, DEGW), jnp.float32),
            pltpu.VMEM_SHARED((n_pad, DEGW), jnp.float32),
        ],
    )
    def deg_kernel(col_hbm, ones_hbm, zeros_hbm, out_hbm, idx_v, ones_v, deg_sh):
        cid = lax.axis_index("c")
        sid = lax.axis_index("s")
        gid = cid * NS + sid
        pltpu.sync_copy(zeros_hbm, deg_sh.at[pl.ds(sid * slab, slab)])
        pltpu.sync_copy(ones_hbm, ones_v)
        pltpu.sync_copy(col_hbm.at[pl.ds(gid * cpt, cpt)], idx_v)
        plsc.subcore_barrier()

        def body(j, carry):
            pltpu.sync_copy(ones_v, deg_sh.at[idx_v.at[j]], add=True)
            return carry

        lax.fori_loop(0, cpt, body, 0)
        plsc.subcore_barrier()
        pltpu.sync_copy(
            deg_sh.at[pl.ds(sid * slab, slab)],
            out_hbm.at[cid, pl.ds(sid * slab, slab)],
        )

    return deg_kernel


def _make_scatter_kernel(n_pad, cpt):
    slab = n_pad // NS

    @functools.partial(
        pl.kernel,
        out_type=jax.ShapeDtypeStruct((NC, n_pad, D), jnp.float32),
        mesh=_mesh(),
        scratch_types=[
            pltpu.VMEM((cpt, CHUNK), jnp.int32),
            pltpu.VMEM((IDXB, CHUNK), jnp.int32),
            pltpu.VMEM((CHUNK, D), jnp.float32),
            pltpu.VMEM((CHUNK, D), jnp.float32),
            pltpu.SemaphoreType.DMA,
            pltpu.SemaphoreType.DMA,
            pltpu.VMEM_SHARED((n_pad, D), jnp.float32),
        ],
    )
    def scatter_kernel(y_hbm, row_hbm, col_hbm, zeros_hbm, out_hbm,
                       rowidx_v, colidx_v, buf_a, buf_b, sem_a, sem_b, acc_sh):
        cid = lax.axis_index("c")
        sid = lax.axis_index("s")
        gid = cid * NS + sid
        pltpu.sync_copy(zeros_hbm, acc_sh.at[pl.ds(sid * slab, slab)])
        pltpu.sync_copy(row_hbm.at[pl.ds(gid * cpt, cpt)], rowidx_v)
        plsc.subcore_barrier()

        bufs = (buf_a, buf_b)
        sems = (sem_a, sem_b)

        def gather(j, b):
            pltpu.async_copy(y_hbm.at[rowidx_v.at[j]], bufs[b], sems[b])

        def gather_wait(j, b):
            pltpu.make_async_copy(y_hbm.at[rowidx_v.at[j]], bufs[b], sems[b]).wait()

        gather(0, 0)
        gather(1, 1)

        def body(j2, carry):
            j = 2 * j2

            @pl.when(lax.rem(j2, IDXB // 2) == 0)
            def _():
                pltpu.sync_copy(
                    col_hbm.at[pl.ds(gid * cpt + (j2 // (IDXB // 2)) * IDXB, IDXB)],
                    colidx_v)

            for b in (0, 1):
                gather_wait(j + b, b)
                pltpu.sync_copy(
                    bufs[b], acc_sh.at[colidx_v.at[lax.rem(j + b, IDXB)]], add=True)

                @pl.when(j + b + 2 < cpt)
                def _():
                    gather(j + b + 2, b)

            return carry

        lax.fori_loop(0, cpt // 2, body, 0)
        plsc.subcore_barrier()
        pltpu.sync_copy(
            acc_sh.at[pl.ds(sid * slab, slab)],
            out_hbm.at[cid, pl.ds(sid * slab, slab)],
        )

    return scatter_kernel


def _linear_body(x_ref, w_ref, degs_ref, y_ref, dis_ref):
    d16 = degs_ref[0] + degs_ref[1]
    deg = jnp.sum(d16, axis=1, keepdims=True) * (1.0 / DEGW) + 1.0
    dis = lax.rsqrt(deg)
    xw = jnp.dot(x_ref[...], w_ref[...], preferred_element_type=jnp.float32)
    y_ref[...] = xw * dis
    dis_ref[...] = dis


def _post_body(n, accs_ref, y_ref, dis_ref, b_ref, gamma_ref, beta_ref, out_ref):
    a = accs_ref[0] + accs_ref[1] + y_ref[...]
    pre = dis_ref[...][:n] * a[:n] + b_ref[...]
    act = jnp.where(pre >= 0, pre, 0.01 * pre)
    mean = jnp.mean(act, axis=0, keepdims=True)
    var = jnp.mean((act - mean) ** 2, axis=0, keepdims=True)
    out_ref[...] = (act - mean) * lax.rsqrt(var + 1e-5) * gamma_ref[...] + beta_ref[...]


def kernel(x, edge_index, W, b, gamma, beta):
    n, d_in = x.shape
    d_out = W.shape[1]
    e = edge_index.shape[1]
    n_pad = ((n + 1) + 8 * NS - 1) // (8 * NS) * (8 * NS)
    cpt = (e + NW * CHUNK - 1) // (NW * CHUNK)
    cpt = (cpt + IDXB - 1) // IDXB * IDXB
    e_pad = NW * CHUNK * cpt

    ei = edge_index.astype(jnp.int32)
    pad = jnp.full((e_pad - e,), n, dtype=jnp.int32)
    row2d = jnp.concatenate([ei[0], pad]).reshape(-1, CHUNK)
    col2d = jnp.concatenate([ei[1], pad]).reshape(-1, CHUNK)
    x_pad = jnp.pad(x, ((0, n_pad - n), (0, 0)))

    slab = n_pad // NS
    ones16 = jnp.ones((CHUNK, DEGW), jnp.float32)
    zeros16 = jnp.zeros((slab, DEGW), jnp.float32)
    zerosd = jnp.zeros((slab, D), jnp.float32)

    degs = _make_deg_kernel(n_pad, cpt)(col2d, ones16, zeros16)

    y, dis = pl.pallas_call(
        _linear_body,
        out_shape=[
            jax.ShapeDtypeStruct((n_pad, d_out), jnp.float32),
            jax.ShapeDtypeStruct((n_pad, 1), jnp.float32),
        ],
    )(x_pad, W, degs)

    accs = _make_scatter_kernel(n_pad, cpt)(y, row2d, col2d, zerosd)

    out = pl.pallas_call(
        functools.partial(_post_body, n),
        out_shape=jax.ShapeDtypeStruct((n, d_out), jnp.float32),
    )(accs, y, dis, b.reshape(1, -1), gamma.reshape(1, -1), beta.reshape(1, -1))
    return out


# speedup vs baseline: 13.8807x; 1.0541x over previous
import functools

import jax
import jax.numpy as jnp
from jax import lax
from jax.experimental import pallas as pl
from jax.experimental.pallas import tpu as pltpu
from jax.experimental.pallas import tpu_sc as plsc

D = 128
CHUNK = 128
NC = 2
NS = 16
NW = NC * NS
IDXB = 16
DEGW = 128


def _mesh():
    return plsc.VectorSubcoreMesh(core_axis_name="c", subcore_axis_name="s")


def _make_deg_kernel(n_pad, cpt):
    slab = n_pad // NS

    @functools.partial(
        pl.kernel,
        out_type=jax.ShapeDtypeStruct((NC, n_pad, DEGW), jnp.float32),
        mesh=_mesh(),
        scratch_types=[
            pltpu.VMEM((cpt, CHUNK), jnp.int32),
            pltpu.VMEM((CHUNK, DEGW), jnp.float32),
            pltpu.VMEM_SHARED((n_pad, DEGW), jnp.float32),
        ],
    )
    def deg_kernel(col_hbm, ones_hbm, zeros_hbm, out_hbm, idx_v, ones_v, deg_sh):
        cid = lax.axis_index("c")
        sid = lax.axis_index("s")
        gid = cid * NS + sid
        pltpu.sync_copy(zeros_hbm, deg_sh.at[pl.ds(sid * slab, slab)])
        pltpu.sync_copy(ones_hbm, ones_v)
        pltpu.sync_copy(col_hbm.at[pl.ds(gid * cpt, cpt)], idx_v)
        plsc.subcore_barrier()

        def body(j, carry):
            pltpu.sync_copy(ones_v, deg_sh.at[idx_v.at[j]], add=True)
            return carry

        lax.fori_loop(0, cpt, body, 0)
        plsc.subcore_barrier()
        pltpu.sync_copy(
            deg_sh.at[pl.ds(sid * slab, slab)],
            out_hbm.at[cid, pl.ds(sid * slab, slab)],
        )

    return deg_kernel


def _make_scatter_kernel(n_pad, cpt_f, cpt_s):
    slab = n_pad // NS

    @functools.partial(
        pl.kernel,
        out_type=jax.ShapeDtypeStruct((NC, n_pad, D), jnp.float32),
        mesh=_mesh(),
        scratch_types=[
            pltpu.VMEM((2, IDXB, CHUNK), jnp.int32),
            pltpu.VMEM((IDXB, CHUNK), jnp.int32),
            pltpu.VMEM((CHUNK, D), jnp.float32),
            pltpu.VMEM((CHUNK, D), jnp.float32),
            pltpu.SemaphoreType.DMA,
            pltpu.SemaphoreType.DMA,
            pltpu.VMEM_SHARED((n_pad, D), jnp.float32),
        ],
    )
    def scatter_kernel(y_hbm, row_hbm, col_hbm, zeros_hbm, out_hbm,
                       ridx_v, cidx_v, buf_a, buf_b, sem_a, sem_b, acc_sh):
        cid = lax.axis_index("c")
        sid = lax.axis_index("s")
        my_cpt = jnp.where(cid == 0, cpt_f, cpt_s)
        my_start = jnp.where(cid == 0, sid * cpt_f, NS * cpt_f + sid * cpt_s)
        pltpu.sync_copy(zeros_hbm, acc_sh.at[pl.ds(sid * slab, slab)])
        pltpu.sync_copy(row_hbm.at[pl.ds(my_start, IDXB)], ridx_v.at[0])
        pltpu.sync_copy(col_hbm.at[pl.ds(my_start, IDXB)], cidx_v)
        plsc.subcore_barrier()

        bufs = (buf_a, buf_b)
        sems = (sem_a, sem_b)

        def ridx_at(j):
            return ridx_v.at[lax.rem(j // IDXB, 2), lax.rem(j, IDXB)]

        def gather(j, b):
            pltpu.async_copy(y_hbm.at[ridx_at(j)], bufs[b], sems[b])

        def gather_wait(j, b):
            pltpu.make_async_copy(y_hbm.at[ridx_at(j)], bufs[b], sems[b]).wait()

        gather(0, 0)
        gather(1, 1)

        def body(j2, carry):
            j = 2 * j2
            blk = j // IDXB

            @pl.when(lax.rem(j, IDXB) == 0)
            def _():
                @pl.when(j > 0)
                def _():
                    pltpu.sync_copy(
                        col_hbm.at[pl.ds(my_start + blk * IDXB, IDXB)], cidx_v)

                @pl.when((blk + 1) * IDXB < my_cpt)
                def _():
                    pltpu.sync_copy(
                        row_hbm.at[pl.ds(my_start + (blk + 1) * IDXB, IDXB)],
                        ridx_v.at[lax.rem(blk + 1, 2)])

            for b in (0, 1):
                jj = j + b
                gather_wait(jj, b)
                pltpu.sync_copy(
                    bufs[b], acc_sh.at[cidx_v.at[lax.rem(jj, IDXB)]], add=True)

                @pl.when(jj + 2 < my_cpt)
                def _():
                    gather(jj + 2, b)

            return carry

        lax.fori_loop(0, my_cpt // 2, body, 0)
        plsc.subcore_barrier()
        pltpu.sync_copy(
            acc_sh.at[pl.ds(sid * slab, slab)],
            out_hbm.at[cid, pl.ds(sid * slab, slab)],
        )

    return scatter_kernel


def _linear_body(x_ref, w_ref, degs_ref, y_ref, dis_ref):
    d16 = degs_ref[0] + degs_ref[1]
    deg = jnp.sum(d16, axis=1, keepdims=True) * (1.0 / DEGW) + 1.0
    dis = lax.rsqrt(deg)
    xw = jnp.dot(x_ref[...], w_ref[...], preferred_element_type=jnp.float32)
    y_ref[...] = xw * dis
    dis_ref[...] = dis


def _post_body(n, accs_ref, y_ref, dis_ref, b_ref, gamma_ref, beta_ref, out_ref):
    a = accs_ref[0] + accs_ref[1] + y_ref[...]
    pre = dis_ref[...][:n] * a[:n] + b_ref[...]
    act = jnp.where(pre >= 0, pre, 0.01 * pre)
    mean = jnp.mean(act, axis=0, keepdims=True)
    var = jnp.mean((act - mean) ** 2, axis=0, keepdims=True)
    out_ref[...] = (act - mean) * lax.rsqrt(var + 1e-5) * gamma_ref[...] + beta_ref[...]


def kernel(x, edge_index, W, b, gamma, beta):
    n, d_in = x.shape
    d_out = W.shape[1]
    e = edge_index.shape[1]
    n_pad = ((n + 1) + 8 * NS - 1) // (8 * NS) * (8 * NS)
    cpt = (e + NW * CHUNK - 1) // (NW * CHUNK)
    cpt = (cpt + IDXB - 1) // IDXB * IDXB
    e_pad = NW * CHUNK * cpt

    ei = edge_index.astype(jnp.int32)
    pad = jnp.full((e_pad - e,), n, dtype=jnp.int32)
    row2d = jnp.concatenate([ei[0], pad]).reshape(-1, CHUNK)
    col2d = jnp.concatenate([ei[1], pad]).reshape(-1, CHUNK)
    x_pad = jnp.pad(x, ((0, n_pad - n), (0, 0)))

    slab = n_pad // NS
    ones16 = jnp.ones((CHUNK, DEGW), jnp.float32)
    zeros16 = jnp.zeros((slab, DEGW), jnp.float32)
    zerosd = jnp.zeros((slab, D), jnp.float32)

    degs = _make_deg_kernel(n_pad, cpt)(col2d, ones16, zeros16)

    y, dis = pl.pallas_call(
        _linear_body,
        out_shape=[
            jax.ShapeDtypeStruct((n_pad, d_out), jnp.float32),
            jax.ShapeDtypeStruct((n_pad, 1), jnp.float32),
        ],
    )(x_pad, W, degs)

    cpt_s = max(8, (2 * cpt) * 2 // 10 // 8 * 8)
    cpt_f = 2 * cpt - cpt_s
    accs = _make_scatter_kernel(n_pad, cpt_f, cpt_s)(y, row2d, col2d, zerosd)

    out = pl.pallas_call(
        functools.partial(_post_body, n),
        out_shape=jax.ShapeDtypeStruct((n, d_out), jnp.float32),
    )(accs, y, dis, b.reshape(1, -1), gamma.reshape(1, -1), beta.reshape(1, -1))
    return out
